# baseline (speedup 1.0000x reference)
import functools

import numpy as np
import jax
import jax.numpy as jnp
from jax.experimental import pallas as pl
from jax.experimental.pallas import tpu as pltpu

_INF_BITS = np.int32(0x7F800000)
_ABS_MASK = np.int32(0x7FFFFFFF)


def _haar(L):
    m = np.zeros((L, L), np.float32)
    inv = np.float32(1.0 / np.sqrt(2.0))
    for k in range(L // 2):
        m[2 * k, k] = inv
        m[2 * k + 1, k] = inv
        m[2 * k, L // 2 + k] = inv
        m[2 * k + 1, L // 2 + k] = -inv
    return m


def _bdiag(block, reps):
    h, w = block.shape
    out = np.zeros((reps * h, reps * w), np.float32)
    for b in range(reps):
        out[b * h:(b + 1) * h, b * w:(b + 1) * w] = block
    return out


def _bdot(a, b_ref):
    return jnp.dot(a.astype(jnp.bfloat16), b_ref[...],
                   preferred_element_type=jnp.float32)


def _bdotl(a_ref, b):
    return jnp.dot(a_ref[...], b.astype(jnp.bfloat16),
                   preferred_element_type=jnp.float32)


def _wnr_body(B, rank, x_ref, raht_ref, rah_ref, rbht_ref, rbh_ref,
              aw_ref, awt_ref, bw_ref, bwt_ref, o_ref):
    xs = x_ref[...].reshape(B * 64, 64)

    y1 = _bdot(_bdotl(raht_ref, xs), aw_ref)
    ll1 = y1.reshape(B, 2, 32, 64)[:, 0].reshape(B * 32, 64)[:, :32]
    y2 = _bdotl(rbht_ref, _bdot(ll1, bw_ref))

    row1 = jax.lax.broadcasted_iota(jnp.int32, (B * 64, 64), 0)
    col1 = jax.lax.broadcasted_iota(jnp.int32, (B * 64, 64), 1)
    is_ll1 = ((row1 % 64) < 32) & (col1 < 32)
    row2 = jax.lax.broadcasted_iota(jnp.int32, (B * 32, 32), 0)
    col2 = jax.lax.broadcasted_iota(jnp.int32, (B * 32, 32), 1)
    is_ll2 = ((row2 % 32) < 16) & (col2 < 16)

    b1 = jax.lax.bitcast_convert_type(y1, jnp.int32) & _ABS_MASK
    b2 = jax.lax.bitcast_convert_type(y2, jnp.int32) & _ABS_MASK
    a2bits = jnp.where(is_ll2, _INF_BITS, b2)

    b1r = b1.reshape(B, 2, 32, 64)
    atop = jnp.concatenate([a2bits, b1r[:, 0].reshape(B * 32, 64)[:, 32:]], axis=1)
    abits = jnp.concatenate([atop, b1r[:, 1].reshape(B * 32, 64)], axis=1)

    abits3 = abits.reshape(B, 32, 128)

    def bisect(_, carry):
        lo_b, hi_b = carry
        mid = lo_b + jax.lax.shift_right_logical(hi_b - lo_b, 1)
        le = (abits3 <= mid).astype(jnp.float32)
        s = le
        for h in (16, 8, 4, 2, 1):
            s = s[:, :h, :] + s[:, h:, :]
        r = s.reshape(B, 128)
        for w in (64, 32, 16, 8, 4, 2, 1):
            r = r[:, :w] + r[:, w:]
        ge = jnp.broadcast_to((r >= jnp.float32(rank)).reshape(B, 1, 1),
                              (B, 1, 128))
        return jnp.where(ge, lo_b, mid + 1), jnp.where(ge, mid, hi_b)

    lo0 = jnp.zeros((B, 1, 128), jnp.int32)
    hi0 = jnp.full((B, 1, 128), _INF_BITS, jnp.int32)
    tbits_v, _ = jax.lax.fori_loop(0, 31, bisect, (lo0, hi0))
    tbits = tbits_v[:, :, :1].reshape(B, 1)

    tb1 = jnp.broadcast_to(tbits.reshape(B, 1, 1), (B, 64, 64)).reshape(B * 64, 64)
    tb2 = jnp.broadcast_to(tbits.reshape(B, 1, 1), (B, 32, 32)).reshape(B * 32, 32)
    y1k = jnp.where((b1 <= tb1) & jnp.logical_not(is_ll1), 0.0, y1)
    y2k = jnp.where((b2 <= tb2) & jnp.logical_not(is_ll2), 0.0, y2)

    ll1r = _bdotl(rbh_ref, _bdot(y2k, bwt_ref))

    y1k_r = y1k.reshape(B, 2, 32, 64)
    top = jnp.concatenate([ll1r, y1k_r[:, 0].reshape(B * 32, 64)[:, 32:]], axis=1)
    y1full = jnp.concatenate(
        [top.reshape(B, 1, 32, 64), y1k_r[:, 1].reshape(B, 1, 32, 64)], axis=1
    ).reshape(B * 64, 64)

    out = _bdotl(rah_ref, _bdot(y1full, awt_ref))
    o_ref[...] = out.reshape(B, 64, 64)


def kernel(images):
    N, C, H, W = images.shape
    assert (H, W) == (64, 64), "kernel specialized for 64x64 planes"
    NC = N * C
    B = 16
    assert NC % B == 0
    G = NC // B

    keep = 16 / 100.0
    s_true = 3 * (H // 2) * (W // 2) + 3 * (H // 4) * (W // 4)
    k = min(int(np.floor((1.0 - keep) * s_true)), s_true - 1)
    rank = k + 1

    ah = _haar(64)
    bh = _haar(32)
    bf = jnp.bfloat16
    RAHT = jnp.asarray(_bdiag(ah.T, B), dtype=bf)
    RAH = jnp.asarray(_bdiag(ah, B), dtype=bf)
    RBHT = jnp.asarray(_bdiag(bh.T, B), dtype=bf)
    RBH = jnp.asarray(_bdiag(bh, B), dtype=bf)
    AW = jnp.asarray(ah, dtype=bf)
    AWT = jnp.asarray(ah.T, dtype=bf)
    BW = jnp.asarray(bh, dtype=bf)
    BWT = jnp.asarray(bh.T, dtype=bf)

    x = images.reshape(NC, 64, 64).astype(jnp.float32)
    cspec = lambda n: pl.BlockSpec((n, n), lambda g: (0, 0))
    out = pl.pallas_call(
        functools.partial(_wnr_body, B, rank),
        grid=(G,),
        in_specs=[
            pl.BlockSpec((B, 64, 64), lambda g: (g, 0, 0)),
            cspec(B * 64), cspec(B * 64), cspec(B * 32), cspec(B * 32),
            cspec(64), cspec(64), cspec(32), cspec(32),
        ],
        out_specs=pl.BlockSpec((B, 64, 64), lambda g: (g, 0, 0)),
        out_shape=jax.ShapeDtypeStruct((NC, 64, 64), jnp.float32),
        compiler_params=pltpu.CompilerParams(
            dimension_semantics=("parallel",)),
    )(x, RAHT, RAH, RBHT, RBH, AW, AWT, BW, BWT)
    return out.reshape(N, C, H, W).astype(images.dtype)


# speedup vs baseline: 3.0665x; 3.0665x over previous
import functools

import numpy as np
import jax
import jax.numpy as jnp
from jax.experimental import pallas as pl
from jax.experimental.pallas import tpu as pltpu

_INF_BITS = np.int32(0x7F800000)
_ABS_MASK = np.int32(0x7FFFFFFF)


def _haar(L):
    m = np.zeros((L, L), np.float32)
    inv = np.float32(1.0 / np.sqrt(2.0))
    for k in range(L // 2):
        m[2 * k, k] = inv
        m[2 * k + 1, k] = inv
        m[2 * k, L // 2 + k] = inv
        m[2 * k + 1, L // 2 + k] = -inv
    return m


def _bdiag(block, reps):
    h, w = block.shape
    out = np.zeros((reps * h, reps * w), np.float32)
    for b in range(reps):
        out[b * h:(b + 1) * h, b * w:(b + 1) * w] = block
    return out


def _bdot(a, b_ref):
    return jnp.dot(a.astype(jnp.bfloat16), b_ref[...],
                   preferred_element_type=jnp.float32)


def _bdotl(a_ref, b):
    return jnp.dot(a_ref[...], b.astype(jnp.bfloat16),
                   preferred_element_type=jnp.float32)


def _wnr_body(B, rank, x_ref, raht_ref, rah_ref, rbht_ref, rbh_ref,
              aw_ref, awt_ref, bw_ref, bwt_ref, o_ref):
    xs = x_ref[...].reshape(B * 64, 64)

    y1 = _bdot(_bdotl(raht_ref, xs), aw_ref)
    ll1 = y1.reshape(B, 2, 32, 64)[:, 0].reshape(B * 32, 64)[:, :32]
    y2 = _bdotl(rbht_ref, _bdot(ll1, bw_ref))

    row1 = jax.lax.broadcasted_iota(jnp.int32, (B * 64, 64), 0)
    col1 = jax.lax.broadcasted_iota(jnp.int32, (B * 64, 64), 1)
    is_ll1 = ((row1 % 64) < 32) & (col1 < 32)
    row2 = jax.lax.broadcasted_iota(jnp.int32, (B * 32, 32), 0)
    col2 = jax.lax.broadcasted_iota(jnp.int32, (B * 32, 32), 1)
    is_ll2 = ((row2 % 32) < 16) & (col2 < 16)

    b1 = jax.lax.bitcast_convert_type(y1, jnp.int32) & _ABS_MASK
    b2 = jax.lax.bitcast_convert_type(y2, jnp.int32) & _ABS_MASK
    a2bits = jnp.where(is_ll2, _INF_BITS, b2)

    b1r = b1.reshape(B, 2, 32, 64)
    atop = jnp.concatenate([a2bits, b1r[:, 0].reshape(B * 32, 64)[:, 32:]], axis=1)
    abits = jnp.concatenate([atop, b1r[:, 1].reshape(B * 32, 64)], axis=1)

    abits3 = abits.reshape(B, 32, 128)

    def bisect(_, carry):
        lo_b, hi_b = carry
        mid = lo_b + jax.lax.shift_right_logical(hi_b - lo_b, 1)
        m1 = jnp.broadcast_to(mid.reshape(B, 1, 1), (B, 32, 128))
        le = (abits3 <= m1).astype(jnp.float32)
        cnt = jnp.sum(jnp.sum(le, axis=1), axis=1, keepdims=True)
        ge = cnt >= jnp.float32(rank)
        return jnp.where(ge, lo_b, mid + 1), jnp.where(ge, mid, hi_b)

    lo0 = jnp.zeros((B, 1), jnp.int32)
    hi0 = jnp.full((B, 1), _INF_BITS, jnp.int32)
    tbits, _ = jax.lax.fori_loop(0, 31, bisect, (lo0, hi0), unroll=2)

    tb1 = jnp.broadcast_to(tbits.reshape(B, 1, 1), (B, 64, 64)).reshape(B * 64, 64)
    tb2 = jnp.broadcast_to(tbits.reshape(B, 1, 1), (B, 32, 32)).reshape(B * 32, 32)
    y1k = jnp.where((b1 <= tb1) & jnp.logical_not(is_ll1), 0.0, y1)
    y2k = jnp.where((b2 <= tb2) & jnp.logical_not(is_ll2), 0.0, y2)

    ll1r = _bdotl(rbh_ref, _bdot(y2k, bwt_ref))

    y1k_r = y1k.reshape(B, 2, 32, 64)
    top = jnp.concatenate([ll1r, y1k_r[:, 0].reshape(B * 32, 64)[:, 32:]], axis=1)
    y1full = jnp.concatenate(
        [top.reshape(B, 1, 32, 64), y1k_r[:, 1].reshape(B, 1, 32, 64)], axis=1
    ).reshape(B * 64, 64)

    out = _bdotl(rah_ref, _bdot(y1full, awt_ref))
    o_ref[...] = out.reshape(B, 64, 64)


def kernel(images):
    N, C, H, W = images.shape
    assert (H, W) == (64, 64), "kernel specialized for 64x64 planes"
    NC = N * C
    B = 16
    assert NC % B == 0
    G = NC // B

    keep = 16 / 100.0
    s_true = 3 * (H // 2) * (W // 2) + 3 * (H // 4) * (W // 4)
    k = min(int(np.floor((1.0 - keep) * s_true)), s_true - 1)
    rank = k + 1

    ah = _haar(64)
    bh = _haar(32)
    bf = jnp.bfloat16
    RAHT = jnp.asarray(_bdiag(ah.T, B), dtype=bf)
    RAH = jnp.asarray(_bdiag(ah, B), dtype=bf)
    RBHT = jnp.asarray(_bdiag(bh.T, B), dtype=bf)
    RBH = jnp.asarray(_bdiag(bh, B), dtype=bf)
    AW = jnp.asarray(ah, dtype=bf)
    AWT = jnp.asarray(ah.T, dtype=bf)
    BW = jnp.asarray(bh, dtype=bf)
    BWT = jnp.asarray(bh.T, dtype=bf)

    x = images.reshape(NC, 64, 64).astype(jnp.float32)
    cspec = lambda n: pl.BlockSpec((n, n), lambda g: (0, 0))
    out = pl.pallas_call(
        functools.partial(_wnr_body, B, rank),
        grid=(G,),
        in_specs=[
            pl.BlockSpec((B, 64, 64), lambda g: (g, 0, 0)),
            cspec(B * 64), cspec(B * 64), cspec(B * 32), cspec(B * 32),
            cspec(64), cspec(64), cspec(32), cspec(32),
        ],
        out_specs=pl.BlockSpec((B, 64, 64), lambda g: (g, 0, 0)),
        out_shape=jax.ShapeDtypeStruct((NC, 64, 64), jnp.float32),
        compiler_params=pltpu.CompilerParams(
            dimension_semantics=("parallel",)),
    )(x, RAHT, RAH, RBHT, RBH, AW, AWT, BW, BWT)
    return out.reshape(N, C, H, W).astype(images.dtype)


# speedup vs baseline: 3.1652x; 1.0322x over previous
import functools

import numpy as np
import jax
import jax.numpy as jnp
from jax.experimental import pallas as pl
from jax.experimental.pallas import tpu as pltpu

_INF_BITS = np.int32(0x7F800000)
_ABS_MASK = np.int32(0x7FFFFFFF)


def _haar(L):
    m = np.zeros((L, L), np.float32)
    inv = np.float32(1.0 / np.sqrt(2.0))
    for k in range(L // 2):
        m[2 * k, k] = inv
        m[2 * k + 1, k] = inv
        m[2 * k, L // 2 + k] = inv
        m[2 * k + 1, L // 2 + k] = -inv
    return m


def _bdiag(block, reps):
    h, w = block.shape
    out = np.zeros((reps * h, reps * w), np.float32)
    for b in range(reps):
        out[b * h:(b + 1) * h, b * w:(b + 1) * w] = block
    return out


def _bdot(a, b_ref):
    return jnp.dot(a.astype(jnp.bfloat16), b_ref[...],
                   preferred_element_type=jnp.float32)


def _bdotl(a_ref, b):
    return jnp.dot(a_ref[...], b.astype(jnp.bfloat16),
                   preferred_element_type=jnp.float32)


def _wnr_body(B, rank, x_ref, raht_ref, rah_ref, rbht_ref, rbh_ref,
              aw_ref, awt_ref, bw_ref, bwt_ref, o_ref):
    xs = x_ref[...].reshape(B * 64, 64)

    y1 = _bdot(_bdotl(raht_ref, xs), aw_ref)
    ll1 = y1.reshape(B, 2, 32, 64)[:, 0].reshape(B * 32, 64)[:, :32]
    y2 = _bdotl(rbht_ref, _bdot(ll1, bw_ref))

    row1 = jax.lax.broadcasted_iota(jnp.int32, (B * 64, 64), 0)
    col1 = jax.lax.broadcasted_iota(jnp.int32, (B * 64, 64), 1)
    is_ll1 = ((row1 % 64) < 32) & (col1 < 32)
    row2 = jax.lax.broadcasted_iota(jnp.int32, (B * 32, 32), 0)
    col2 = jax.lax.broadcasted_iota(jnp.int32, (B * 32, 32), 1)
    is_ll2 = ((row2 % 32) < 16) & (col2 < 16)

    b1 = jax.lax.bitcast_convert_type(y1, jnp.int32) & _ABS_MASK
    b2 = jax.lax.bitcast_convert_type(y2, jnp.int32) & _ABS_MASK
    a2bits = jnp.where(is_ll2, _INF_BITS, b2)

    b1r = b1.reshape(B, 2, 32, 64)
    atop = jnp.concatenate([a2bits, b1r[:, 0].reshape(B * 32, 64)[:, 32:]], axis=1)
    abits = jnp.concatenate([atop, b1r[:, 1].reshape(B * 32, 64)], axis=1)

    abits3 = abits.reshape(B, 32, 128)

    def bisect(_, carry):
        lo_b, hi_b = carry
        mid = lo_b + jax.lax.shift_right_logical(hi_b - lo_b, 1)
        m1 = jnp.broadcast_to(mid.reshape(B, 1, 1), (B, 32, 128))
        le = (abits3 <= m1).astype(jnp.float32)
        cnt = jnp.sum(jnp.sum(le, axis=1), axis=1, keepdims=True)
        ge = cnt >= jnp.float32(rank)
        return jnp.where(ge, lo_b, mid + 1), jnp.where(ge, mid, hi_b)

    lo0 = jnp.zeros((B, 1), jnp.int32)
    hi0 = jnp.full((B, 1), _INF_BITS, jnp.int32)
    tbits, _ = jax.lax.fori_loop(0, 31, bisect, (lo0, hi0), unroll=4)

    tb1 = jnp.broadcast_to(tbits.reshape(B, 1, 1), (B, 64, 64)).reshape(B * 64, 64)
    tb2 = jnp.broadcast_to(tbits.reshape(B, 1, 1), (B, 32, 32)).reshape(B * 32, 32)
    y1k = jnp.where((b1 <= tb1) & jnp.logical_not(is_ll1), 0.0, y1)
    y2k = jnp.where((b2 <= tb2) & jnp.logical_not(is_ll2), 0.0, y2)

    ll1r = _bdotl(rbh_ref, _bdot(y2k, bwt_ref))

    y1k_r = y1k.reshape(B, 2, 32, 64)
    top = jnp.concatenate([ll1r, y1k_r[:, 0].reshape(B * 32, 64)[:, 32:]], axis=1)
    y1full = jnp.concatenate(
        [top.reshape(B, 1, 32, 64), y1k_r[:, 1].reshape(B, 1, 32, 64)], axis=1
    ).reshape(B * 64, 64)

    out = _bdotl(rah_ref, _bdot(y1full, awt_ref))
    o_ref[...] = out.reshape(B, 64, 64)


def kernel(images):
    N, C, H, W = images.shape
    assert (H, W) == (64, 64), "kernel specialized for 64x64 planes"
    NC = N * C
    B = 16
    assert NC % B == 0
    G = NC // B

    keep = 16 / 100.0
    s_true = 3 * (H // 2) * (W // 2) + 3 * (H // 4) * (W // 4)
    k = min(int(np.floor((1.0 - keep) * s_true)), s_true - 1)
    rank = k + 1

    ah = _haar(64)
    bh = _haar(32)
    bf = jnp.bfloat16
    RAHT = jnp.asarray(_bdiag(ah.T, B), dtype=bf)
    RAH = jnp.asarray(_bdiag(ah, B), dtype=bf)
    RBHT = jnp.asarray(_bdiag(bh.T, B), dtype=bf)
    RBH = jnp.asarray(_bdiag(bh, B), dtype=bf)
    AW = jnp.asarray(ah, dtype=bf)
    AWT = jnp.asarray(ah.T, dtype=bf)
    BW = jnp.asarray(bh, dtype=bf)
    BWT = jnp.asarray(bh.T, dtype=bf)

    x = images.reshape(NC, 64, 64).astype(jnp.float32)
    cspec = lambda n: pl.BlockSpec((n, n), lambda g: (0, 0))
    out = pl.pallas_call(
        functools.partial(_wnr_body, B, rank),
        grid=(G,),
        in_specs=[
            pl.BlockSpec((B, 64, 64), lambda g: (g, 0, 0)),
            cspec(B * 64), cspec(B * 64), cspec(B * 32), cspec(B * 32),
            cspec(64), cspec(64), cspec(32), cspec(32),
        ],
        out_specs=pl.BlockSpec((B, 64, 64), lambda g: (g, 0, 0)),
        out_shape=jax.ShapeDtypeStruct((NC, 64, 64), jnp.float32),
        compiler_params=pltpu.CompilerParams(
            dimension_semantics=("parallel",)),
    )(x, RAHT, RAH, RBHT, RBH, AW, AWT, BW, BWT)
    return out.reshape(N, C, H, W).astype(images.dtype)
